# Initial kernel scaffold; baseline (speedup 1.0000x reference)
import jax
import jax.numpy as jnp
from jax.experimental import pallas as pl


def kernel(var_val, var_type, pred_table, bool_table, gamma_p, beta_p, gamma_b, beta_b):
    raise NotImplementedError("write your pallas kernel here")


# speedup vs baseline: 18.5642x; 18.5642x over previous
import functools

import jax
import jax.numpy as jnp
from jax import lax
from jax.experimental import pallas as pl
from jax.experimental.pallas import tpu as pltpu
from jax.experimental.pallas import tpu_sc as plsc

_VOCAB = 100000
_D = 64
_EPS = 1e-5


_ROWS_PER_BLOCK = 1000


def _table_body(pred_ref, bool_ref, gp_ref, bp_ref, gb_ref, bb_ref, out_ref):
    x = pred_ref[...]
    m = jnp.mean(x, axis=-1, keepdims=True)
    v = jnp.mean((x - m) ** 2, axis=-1, keepdims=True)
    xn = (x - m) / jnp.sqrt(v + _EPS) * gp_ref[...] + bp_ref[...]
    b = bool_ref[...]
    bm = jnp.mean(b, axis=-1, keepdims=True)
    bv = jnp.mean((b - bm) ** 2, axis=-1, keepdims=True)
    bn = (b - bm) / jnp.sqrt(bv + _EPS) * gb_ref[...] + bb_ref[...]
    out_ref[0] = xn * bn[0:1]
    out_ref[1] = xn * bn[1:2]


def _build_combined_table(pred_table, bool_table, gamma_p, beta_p, gamma_b, beta_b):
    r = _ROWS_PER_BLOCK
    comb3 = pl.pallas_call(
        _table_body,
        grid=(_VOCAB // r,),
        in_specs=[
            pl.BlockSpec((r, _D), lambda i: (i, 0)),
            pl.BlockSpec((2, _D), lambda i: (0, 0)),
            pl.BlockSpec((1, _D), lambda i: (0, 0)),
            pl.BlockSpec((1, _D), lambda i: (0, 0)),
            pl.BlockSpec((1, _D), lambda i: (0, 0)),
            pl.BlockSpec((1, _D), lambda i: (0, 0)),
        ],
        out_specs=pl.BlockSpec((2, r, _D), lambda i: (0, i, 0)),
        out_shape=jax.ShapeDtypeStruct((2, _VOCAB, _D), jnp.float32),
    )(
        pred_table,
        bool_table,
        gamma_p.reshape(1, _D),
        beta_p.reshape(1, _D),
        gamma_b.reshape(1, _D),
        beta_b.reshape(1, _D),
    )
    return comb3.reshape(2 * _VOCAB, _D)


_CHUNK = 512
_GATHER = 128
_K = _CHUNK // _GATHER


def _make_gather_kernel(n_rows, rows_per_w, nc):
    n_chunks = rows_per_w // _CHUNK
    mesh = plsc.VectorSubcoreMesh(core_axis_name="c", subcore_axis_name="s")

    @functools.partial(
        pl.kernel,
        out_type=jax.ShapeDtypeStruct((n_rows, _D), jnp.float32),
        mesh=mesh,
        compiler_params=pltpu.CompilerParams(use_tc_tiling_on_sc=False),
        scratch_types=[
            pltpu.VMEM((_CHUNK,), jnp.int32),
            pltpu.VMEM((_CHUNK,), jnp.int32),
            pltpu.VMEM((_CHUNK,), jnp.int32),
            pltpu.VMEM((_CHUNK, _D), jnp.float32),
            pltpu.SemaphoreType.DMA,
        ],
    )
    def _gather(vv_hbm, vt_hbm, comb_hbm, out_hbm, vv_v, vt_v, idx_v, rows_v, sem):
        wid = lax.axis_index("s") * nc + lax.axis_index("c")
        base = wid * rows_per_w

        def chunk_body(ci, carry):
            start = base + ci * _CHUNK
            pltpu.sync_copy(vv_hbm.at[pl.ds(start, _CHUNK)], vv_v)
            pltpu.sync_copy(vt_hbm.at[pl.ds(start, _CHUNK)], vt_v)

            def idx_body(j, c):
                o = j * 16
                idx_v[pl.ds(o, 16)] = vv_v[pl.ds(o, 16)] * _VOCAB + vt_v[pl.ds(o, 16)]
                return c

            lax.fori_loop(0, _CHUNK // 16, idx_body, 0)

            copies = [
                pltpu.async_copy(
                    comb_hbm.at[idx_v.at[pl.ds(k * _GATHER, _GATHER)]],
                    rows_v.at[pl.ds(k * _GATHER, _GATHER)],
                    sem,
                )
                for k in range(_K)
            ]
            for cp in copies:
                cp.wait()
            pltpu.sync_copy(rows_v, out_hbm.at[pl.ds(start, _CHUNK)])
            return carry

        lax.fori_loop(0, n_chunks, chunk_body, 0)

    return _gather


def kernel(var_val, var_type, pred_table, bool_table, gamma_p, beta_p, gamma_b, beta_b):
    b, f = var_val.shape
    n_rows = b * f

    comb = _build_combined_table(pred_table, bool_table, gamma_p, beta_p, gamma_b, beta_b)

    info = plsc.get_sparse_core_info()
    nw = info.num_cores * info.num_subcores
    rows_per_w = n_rows // nw

    vv = var_val.reshape(n_rows).astype(jnp.int32)
    vt = var_type.reshape(n_rows).astype(jnp.int32)

    gather = _make_gather_kernel(n_rows, rows_per_w, info.num_cores)
    out = gather(vv, vt, comb)
    return out.reshape(b, f, _D)


# speedup vs baseline: 20.3424x; 1.0958x over previous
import functools

import jax
import jax.numpy as jnp
from jax import lax
from jax.experimental import pallas as pl
from jax.experimental.pallas import tpu as pltpu
from jax.experimental.pallas import tpu_sc as plsc

_VOCAB = 100000
_D = 64
_EPS = 1e-5


_ROWS_PER_BLOCK = 1000


def _table_body(pred_ref, bool_ref, gp_ref, bp_ref, gb_ref, bb_ref, out_ref):
    x = pred_ref[...]
    m = jnp.mean(x, axis=-1, keepdims=True)
    v = jnp.mean((x - m) ** 2, axis=-1, keepdims=True)
    xn = (x - m) / jnp.sqrt(v + _EPS) * gp_ref[...] + bp_ref[...]
    b = bool_ref[...]
    bm = jnp.mean(b, axis=-1, keepdims=True)
    bv = jnp.mean((b - bm) ** 2, axis=-1, keepdims=True)
    bn = (b - bm) / jnp.sqrt(bv + _EPS) * gb_ref[...] + bb_ref[...]
    out_ref[...] = jnp.concatenate([xn * bn[0:1], xn * bn[1:2]], axis=-1)


def _build_combined_table(pred_table, bool_table, gamma_p, beta_p, gamma_b, beta_b):
    r = _ROWS_PER_BLOCK
    comb2 = pl.pallas_call(
        _table_body,
        grid=(_VOCAB // r,),
        in_specs=[
            pl.BlockSpec((r, _D), lambda i: (i, 0)),
            pl.BlockSpec((2, _D), lambda i: (0, 0)),
            pl.BlockSpec((1, _D), lambda i: (0, 0)),
            pl.BlockSpec((1, _D), lambda i: (0, 0)),
            pl.BlockSpec((1, _D), lambda i: (0, 0)),
            pl.BlockSpec((1, _D), lambda i: (0, 0)),
        ],
        out_specs=pl.BlockSpec((r, 2 * _D), lambda i: (i, 0)),
        out_shape=jax.ShapeDtypeStruct((_VOCAB, 2 * _D), jnp.float32),
    )(
        pred_table,
        bool_table,
        gamma_p.reshape(1, _D),
        beta_p.reshape(1, _D),
        gamma_b.reshape(1, _D),
        beta_b.reshape(1, _D),
    )
    return comb2.reshape(2 * _VOCAB, _D)


_NB = 8


def _make_gather_kernel(b_dim, f_dim, nc, ns):
    nw = nc * ns
    b_per_w = b_dim // nw
    n_chunks = b_per_w // _NB
    rows_per_chunk = _NB * f_dim
    mesh = plsc.VectorSubcoreMesh(core_axis_name="c", subcore_axis_name="s")

    @functools.partial(
        pl.kernel,
        out_type=jax.ShapeDtypeStruct((b_dim, f_dim, _D), jnp.float32),
        mesh=mesh,
        compiler_params=pltpu.CompilerParams(use_tc_tiling_on_sc=False),
        scratch_types=[
            pltpu.VMEM((rows_per_chunk + 16,), jnp.int32),
            pltpu.VMEM((rows_per_chunk + 16,), jnp.int32),
            pltpu.VMEM((_NB, f_dim), jnp.int32),
            pltpu.VMEM((_NB, f_dim, _D), jnp.float32),
            pltpu.SemaphoreType.DMA,
        ],
    )
    def _gather(vv_hbm, vt_hbm, comb_hbm, out_hbm, vv_v, vt_v, idx_v, rows_v, sem):
        wid = lax.axis_index("s") * nc + lax.axis_index("c")
        b_base = wid * b_per_w

        tail0 = f_dim - 16

        def chunk_body(ci, carry):
            b0 = b_base + ci * _NB
            row0 = b0 * f_dim
            pltpu.sync_copy(vv_hbm.at[pl.ds(row0, rows_per_chunk)],
                            vv_v.at[pl.ds(0, rows_per_chunk)])
            pltpu.sync_copy(vt_hbm.at[pl.ds(row0, rows_per_chunk)],
                            vt_v.at[pl.ds(0, rows_per_chunk)])

            for j in range(_NB):
                for o in list(range(0, tail0, 16)) + [tail0]:
                    p = j * f_dim + o
                    idx_v[j, pl.ds(o, 16)] = (
                        vt_v[pl.ds(p, 16)] * 2 + vv_v[pl.ds(p, 16)]
                    )

            copies = [
                pltpu.async_copy(
                    comb_hbm.at[idx_v.at[j]],
                    rows_v.at[j],
                    sem,
                )
                for j in range(_NB)
            ]
            for cp in copies:
                cp.wait()
            pltpu.sync_copy(rows_v, out_hbm.at[pl.ds(b0, _NB)])
            return carry

        lax.fori_loop(0, n_chunks, chunk_body, 0)

    return _gather


def kernel(var_val, var_type, pred_table, bool_table, gamma_p, beta_p, gamma_b, beta_b):
    b, f = var_val.shape
    n_rows = b * f

    comb = _build_combined_table(pred_table, bool_table, gamma_p, beta_p, gamma_b, beta_b)

    info = plsc.get_sparse_core_info()

    vv = var_val.reshape(n_rows).astype(jnp.int32)
    vt = var_type.reshape(n_rows).astype(jnp.int32)

    gather = _make_gather_kernel(b, f, info.num_cores, info.num_subcores)
    return gather(vv, vt, comb)


# speedup vs baseline: 22.0265x; 1.0828x over previous
import functools

import jax
import jax.numpy as jnp
from jax import lax
from jax.experimental import pallas as pl
from jax.experimental.pallas import tpu as pltpu
from jax.experimental.pallas import tpu_sc as plsc

_VOCAB = 100000
_D = 64
_EPS = 1e-5


_ROWS_PER_BLOCK = 1000


def _table_body(pred_ref, bool_ref, gp_ref, bp_ref, gb_ref, bb_ref, out_ref):
    x = pred_ref[...]
    m = jnp.mean(x, axis=-1, keepdims=True)
    v = jnp.mean((x - m) ** 2, axis=-1, keepdims=True)
    xn = (x - m) / jnp.sqrt(v + _EPS) * gp_ref[...] + bp_ref[...]
    b = bool_ref[...]
    bm = jnp.mean(b, axis=-1, keepdims=True)
    bv = jnp.mean((b - bm) ** 2, axis=-1, keepdims=True)
    bn = (b - bm) / jnp.sqrt(bv + _EPS) * gb_ref[...] + bb_ref[...]
    out_ref[...] = jnp.concatenate([xn * bn[0:1], xn * bn[1:2]], axis=-1)


def _build_combined_table(pred_table, bool_table, gamma_p, beta_p, gamma_b, beta_b):
    r = _ROWS_PER_BLOCK
    comb2 = pl.pallas_call(
        _table_body,
        grid=(_VOCAB // r,),
        in_specs=[
            pl.BlockSpec((r, _D), lambda i: (i, 0)),
            pl.BlockSpec((2, _D), lambda i: (0, 0)),
            pl.BlockSpec((1, _D), lambda i: (0, 0)),
            pl.BlockSpec((1, _D), lambda i: (0, 0)),
            pl.BlockSpec((1, _D), lambda i: (0, 0)),
            pl.BlockSpec((1, _D), lambda i: (0, 0)),
        ],
        out_specs=pl.BlockSpec((r, 2 * _D), lambda i: (i, 0)),
        out_shape=jax.ShapeDtypeStruct((_VOCAB, 2 * _D), jnp.float32),
    )(
        pred_table,
        bool_table,
        gamma_p.reshape(1, _D),
        beta_p.reshape(1, _D),
        gamma_b.reshape(1, _D),
        beta_b.reshape(1, _D),
    )
    return comb2.reshape(2 * _VOCAB, _D)


_FG = 8
_GATHER = 128


def _make_gather_kernel(b_dim, f_dim, f_pad, nc, ns):
    nw = nc * ns
    b_per_w = b_dim // nw
    n_blocks = f_pad // _FG
    n_tr = b_per_w // _GATHER
    mesh = plsc.VectorSubcoreMesh(core_axis_name="c", subcore_axis_name="s")

    @functools.partial(
        pl.kernel,
        out_type=jax.ShapeDtypeStruct((f_dim, b_dim, _D), jnp.float32),
        mesh=mesh,
        compiler_params=pltpu.CompilerParams(use_tc_tiling_on_sc=False),
        scratch_types=[
            pltpu.VMEM((_FG, b_per_w), jnp.int32),
            pltpu.VMEM((_FG, b_per_w), jnp.int32),
            pltpu.VMEM((b_per_w,), jnp.int32),
            pltpu.VMEM((b_per_w, _D), jnp.float32),
            pltpu.SemaphoreType.DMA,
        ],
    )
    def _gather(vvT_hbm, vtT_hbm, comb_hbm, out_hbm, vv_v, vt_v, idx_v, rows_v, sem):
        wid = lax.axis_index("s") * nc + lax.axis_index("c")
        b0 = wid * b_per_w

        def blk_body(blk, carry):
            f0 = blk * _FG
            pltpu.sync_copy(vvT_hbm.at[pl.ds(f0, _FG), pl.ds(b0, b_per_w)], vv_v)
            pltpu.sync_copy(vtT_hbm.at[pl.ds(f0, _FG), pl.ds(b0, b_per_w)], vt_v)

            for fl in range(_FG):
                @pl.when(f0 + fl < f_dim)
                def _():
                    def idx_body(j, c):
                        o = j * 16
                        idx_v[pl.ds(o, 16)] = (
                            vt_v[fl, pl.ds(o, 16)] * 2 + vv_v[fl, pl.ds(o, 16)]
                        )
                        return c

                    lax.fori_loop(0, b_per_w // 16, idx_body, 0)
                    copies = [
                        pltpu.async_copy(
                            comb_hbm.at[idx_v.at[pl.ds(k * _GATHER, _GATHER)]],
                            rows_v.at[pl.ds(k * _GATHER, _GATHER)],
                            sem,
                        )
                        for k in range(n_tr)
                    ]
                    for cp in copies:
                        cp.wait()
                    pltpu.sync_copy(
                        rows_v, out_hbm.at[f0 + fl, pl.ds(b0, b_per_w), :]
                    )

            return carry

        lax.fori_loop(0, n_blocks, blk_body, 0)

    return _gather


def kernel(var_val, var_type, pred_table, bool_table, gamma_p, beta_p, gamma_b, beta_b):
    b, f = var_val.shape

    comb = _build_combined_table(pred_table, bool_table, gamma_p, beta_p, gamma_b, beta_b)

    info = plsc.get_sparse_core_info()

    f_pad = ((f + _FG - 1) // _FG) * _FG
    vvT = jnp.transpose(var_val.astype(jnp.int32))
    vtT = jnp.transpose(var_type.astype(jnp.int32))
    pad = ((0, f_pad - f), (0, 0))
    vvT = jnp.pad(vvT, pad)
    vtT = jnp.pad(vtT, pad)

    gather = _make_gather_kernel(b, f, f_pad, info.num_cores, info.num_subcores)
    xfb = gather(vvT, vtT, comb)
    return jnp.transpose(xfb, (1, 0, 2))
